# baseline (speedup 1.0000x reference)
import functools

import jax
import jax.numpy as jnp
from jax import lax
from jax.experimental import pallas as pl
from jax.experimental.pallas import tpu as pltpu
from jax.experimental.pallas import tpu_sc as plsc

B, L = 1024, 200
N = B * L
W = 100001
NB1 = 8
NB2 = 8
D = NB1 + NB2
MROWS = 12 * NB1
WROWS = 52 * NB2
TPR = 782
GSTRIDE = TPR * 1024

NC, NS = 2, 16
NWK = NC * NS
PER_W = N // NWK
C = 1600
CHUNKS = PER_W // C
VC = C // 2


BLK = 2048
NFULL = 48
TAILC = 99968 - NFULL * BLK
TAILW = (TAILC // 128 + 1) * 1024


def _retile_body(month_hbm, week_hbm, mrem_hbm, wrem_hbm, mflat, wflat,
                 buf0, buf1, rbuf, sin0, sin1, sout0, sout1, srem):
    wid = lax.axis_index("s") * NC + lax.axis_index("c")
    bufs = (buf0, buf1)
    sins, souts = (sin0, sin1), (sout0, sout1)

    def in_cp(tbl, m, b, s, cols):
        return pltpu.make_async_copy(
            tbl.at[pl.ds(8 * m, 8), pl.ds(pl.multiple_of(b * BLK, 128), cols)],
            bufs[s].at[:, pl.ds(0, cols)],
            sins[s],
        )

    def out_tile(dst, m, b, s, t):
        return pltpu.make_async_copy(
            bufs[s].at[:, pl.ds(pl.multiple_of(t * 128, 128), 128)],
            dst.at[m * TPR + b * (BLK // 128) + t],
            souts[s],
        )

    def run_group(tbl, rem, dst, m):
        in_cp(tbl, m, 0, 0, BLK).start()

        def slot_body(s, b):
            in_cp(tbl, m, b, s, BLK).wait()

            @pl.when(b + 1 < NFULL)
            def _():
                in_cp(tbl, m, b + 1, 1 - s, BLK).start()

            @pl.when(b + 1 == NFULL)
            def _():
                in_cp(tbl, m, NFULL, 1 - s, TAILC).start()

            @pl.when(b >= 2)
            def _():
                for t in range(16):
                    out_tile(dst, m, b - 2, s, t).wait()

            for t in range(16):
                out_tile(dst, m, b, s, t).start()

        def block_body(b, _):
            @pl.when((b & 1) == 0)
            def _():
                slot_body(0, b)

            @pl.when((b & 1) == 1)
            def _():
                slot_body(1, b)

            return 0

        lax.fori_loop(0, NFULL, block_body, 0)

        s = NFULL & 1
        crm = pltpu.make_async_copy(rem.at[pl.ds(8 * m, 8), :], rbuf, srem)
        crm.start()
        in_cp(tbl, m, NFULL, s, TAILC).wait()
        for t in range(16):
            out_tile(dst, m, NFULL - 2, s, t).wait()
        for t in range(TAILC // 128):
            out_tile(dst, m, NFULL, s, t).start()
        crm.wait()
        ct = pltpu.make_async_copy(rbuf, dst.at[m * TPR + 781], srem)
        ct.start()
        for t in range(16):
            out_tile(dst, m, NFULL - 1, 1 - s, t).wait()
        for t in range(TAILC // 128):
            out_tile(dst, m, NFULL, s, t).wait()
        ct.wait()

    @pl.when(wid < MROWS // 8 // 2)
    def _():
        def j_body(j, _):
            run_group(month_hbm, mrem_hbm, mflat, 2 * wid + j)
            return 0
        lax.fori_loop(0, 2, j_body, 0)

    @pl.when(wid >= MROWS // 8 // 2)
    def _():
        def j_body(j, _):
            run_group(week_hbm, wrem_hbm, wflat, 2 * (wid - MROWS // 8 // 2) + j)
            return 0
        lax.fori_loop(0, 2, j_body, 0)


def _gather_body(item_hbm, t1_hbm, t2_hbm, month_flat, week_flat, out_hbm,
                 item_v, t1_v, t2_v, midx_v, widx_v, sbuf,
                 sem_m, sem_w):
    wid = lax.axis_index("s") * NC + lax.axis_index("c")
    s0 = jnp.where(wid < 24, 6 * wid, 144 + 7 * (wid - 24))
    ns = jnp.where(wid < 24, 6, 7)

    pltpu.sync_copy(item_hbm.at[pl.ds(1024 * s0, 7168)], item_v)
    pltpu.sync_copy(t1_hbm.at[pl.ds(1024 * s0, 7168)], t1_v)
    pltpu.sync_copy(t2_hbm.at[pl.ds(1024 * s0, 7168)], t2_v)

    def slab(j, _):
        jb = 1024 * j

        def build(g, _):
            o = jb + 16 * g
            it16 = item_v[pl.ds(o, 16)]
            cbase = lax.shift_left(lax.shift_right_logical(it16, 7), 10) + (it16 & 127)
            mb16 = t1_v[pl.ds(o, 16)] * GSTRIDE + cbase
            wb16 = t2_v[pl.ds(o, 16)] * GSTRIDE + cbase
            vb = lax.shift_left(lax.shift_right_logical(g, 3), 10) + 16 * (g & 7)
            for dr in range(8):
                midx_v[pl.ds(vb + dr * 128, 16)] = mb16 + dr * 128
                widx_v[pl.ds(vb + dr * 128, 16)] = wb16 + dr * 128
            return 0

        lax.fori_loop(0, 64, build, 0)

        cpm = pltpu.make_async_copy(month_flat.at[midx_v], sbuf.at[pl.ds(0, 8192)], sem_m)
        cpw = pltpu.make_async_copy(week_flat.at[widx_v], sbuf.at[pl.ds(8192, 8192)], sem_w)
        cpm.start()
        cpw.start()
        cpm.wait()
        cpw.wait()

        pltpu.sync_copy(sbuf, out_hbm.at[pl.ds(16384 * (s0 + j), 16384)])
        return 0

    lax.fori_loop(0, ns, slab, 0)


@jax.jit
def _popularity_encode(item_flat, t1_flat, t2_flat, month_tbl, week_tbl,
                       mrem, wrem):
    mesh = plsc.VectorSubcoreMesh(core_axis_name="c", subcore_axis_name="s")
    retile = pl.kernel(
        _retile_body,
        out_type=(
            jax.ShapeDtypeStruct(((MROWS // 8) * TPR, 8, 128), jnp.float32),
            jax.ShapeDtypeStruct(((WROWS // 8) * TPR, 8, 128), jnp.float32),
        ),
        mesh=mesh,
        scratch_types=[
            pltpu.VMEM((8, BLK), jnp.float32),
            pltpu.VMEM((8, BLK), jnp.float32),
            pltpu.VMEM((8, 128), jnp.float32),
            pltpu.SemaphoreType.DMA,
            pltpu.SemaphoreType.DMA,
            pltpu.SemaphoreType.DMA,
            pltpu.SemaphoreType.DMA,
            pltpu.SemaphoreType.DMA,
        ],
        name="popularity_retile_sc",
    )
    m3, w3 = retile(month_tbl, week_tbl, mrem, wrem)
    month_flat = m3.reshape(-1)
    week_flat = w3.reshape(-1)
    gather = pl.kernel(
        _gather_body,
        out_type=jax.ShapeDtypeStruct((N * D,), jnp.float32),
        mesh=mesh,
        scratch_types=[
            pltpu.VMEM((7168,), jnp.int32),
            pltpu.VMEM((7168,), jnp.int32),
            pltpu.VMEM((7168,), jnp.int32),
            pltpu.VMEM((8192,), jnp.int32),
            pltpu.VMEM((8192,), jnp.int32),
            pltpu.VMEM((16384,), jnp.float32),
            pltpu.SemaphoreType.DMA,
            pltpu.SemaphoreType.DMA,
        ],
        name="popularity_encoding_sc",
    )
    return gather(item_flat, t1_flat, t2_flat, month_flat, week_flat)


def kernel(log_seqs, time1_seqs, time2_seqs, month_pop_table, week_pop_table):
    item_flat = log_seqs.T.reshape(-1).astype(jnp.int32)
    t1_flat = time1_seqs.T.reshape(-1).astype(jnp.int32)
    t2_flat = time2_seqs.T.reshape(-1).astype(jnp.int32)
    pad = 128 - (W - 99968)
    mrem = jnp.pad(month_pop_table[:, 99968:], ((0, 0), (0, pad)))
    wrem = jnp.pad(week_pop_table[:, 99968:], ((0, 0), (0, pad)))
    out = _popularity_encode(item_flat, t1_flat, t2_flat,
                             month_pop_table, week_pop_table, mrem, wrem)
    return (out.reshape(L, 2, 8, 8, 128)
            .transpose(2, 4, 0, 1, 3)
            .reshape(B, L, D))


# speedup vs baseline: 1.0271x; 1.0271x over previous
import functools

import jax
import jax.numpy as jnp
from jax import lax
from jax.experimental import pallas as pl
from jax.experimental.pallas import tpu as pltpu
from jax.experimental.pallas import tpu_sc as plsc

B, L = 1024, 200
N = B * L
W = 100001
NB1 = 8
NB2 = 8
D = NB1 + NB2
MROWS = 12 * NB1
WROWS = 52 * NB2
TPR = 782
GSTRIDE = TPR * 1024

NC, NS = 2, 16
NWK = NC * NS
PER_W = N // NWK
C = 1600
CHUNKS = PER_W // C
VC = C // 2


BLK = 2048
NFULL = 48
TAILC = 99968 - NFULL * BLK
TAILW = (TAILC // 128 + 1) * 1024


def _retile_body(month_hbm, week_hbm, mrem_hbm, wrem_hbm, mflat, wflat,
                 buf0, buf1, stage0, stage1, rbuf,
                 sin0, sin1, sout0, sout1, srem):
    wid = lax.axis_index("s") * NC + lax.axis_index("c")
    bufs, stages = (buf0, buf1), (stage0, stage1)
    sins, souts = (sin0, sin1), (sout0, sout1)

    def in_cp(tbl, m, b, s, cols):
        return pltpu.make_async_copy(
            tbl.at[pl.ds(8 * m, 8), pl.ds(pl.multiple_of(b * BLK, 128), cols)],
            bufs[s].at[:, pl.ds(0, cols)],
            sins[s],
        )

    def out_cp(dst, m, b, s, words):
        return pltpu.make_async_copy(
            stages[s].at[pl.ds(0, words)],
            dst.at[pl.ds(m * GSTRIDE + b * (BLK * 8), words)],
            souts[s],
        )

    def redistribute(s, ntiles):
        def tile_body(t, _):
            for r in range(8):
                for kk in range(8):
                    stages[s][pl.ds(t * 1024 + r * 128 + 16 * kk, 16)] = (
                        bufs[s][r, pl.ds(t * 128 + 16 * kk, 16)])
            return 0
        lax.fori_loop(0, ntiles, tile_body, 0)

    def run_group(tbl, rem, dst, m):
        in_cp(tbl, m, 0, 0, BLK).start()

        def slot_body(s, b):
            in_cp(tbl, m, b, s, BLK).wait()

            @pl.when(b + 1 < NFULL)
            def _():
                in_cp(tbl, m, b + 1, 1 - s, BLK).start()

            @pl.when(b + 1 == NFULL)
            def _():
                in_cp(tbl, m, NFULL, 1 - s, TAILC).start()

            @pl.when(b >= 2)
            def _():
                out_cp(dst, m, b - 2, s, BLK * 8).wait()

            redistribute(s, 16)
            out_cp(dst, m, b, s, BLK * 8).start()

        def block_body(b, _):
            @pl.when((b & 1) == 0)
            def _():
                slot_body(0, b)

            @pl.when((b & 1) == 1)
            def _():
                slot_body(1, b)

            return 0

        lax.fori_loop(0, NFULL, block_body, 0)

        s = NFULL & 1
        crm = pltpu.make_async_copy(rem.at[pl.ds(8 * m, 8), :], rbuf, srem)
        crm.start()
        in_cp(tbl, m, NFULL, s, TAILC).wait()
        out_cp(dst, m, NFULL - 2, s, BLK * 8).wait()
        redistribute(s, TAILC // 128)
        crm.wait()
        for r in range(8):
            for kk in range(8):
                stages[s][pl.ds((TAILC // 128) * 1024 + r * 128 + 16 * kk, 16)] = (
                    rbuf[r, pl.ds(16 * kk, 16)])
        out_cp(dst, m, NFULL, s, TAILW).start()
        out_cp(dst, m, NFULL - 1, 1 - s, BLK * 8).wait()
        out_cp(dst, m, NFULL, s, TAILW).wait()

    @pl.when(wid < MROWS // 8 // 2)
    def _():
        def j_body(j, _):
            run_group(month_hbm, mrem_hbm, mflat, 2 * wid + j)
            return 0
        lax.fori_loop(0, 2, j_body, 0)

    @pl.when(wid >= MROWS // 8 // 2)
    def _():
        def j_body(j, _):
            run_group(week_hbm, wrem_hbm, wflat, 2 * (wid - MROWS // 8 // 2) + j)
            return 0
        lax.fori_loop(0, 2, j_body, 0)


def _gather_body(item_hbm, t1_hbm, t2_hbm, month_flat, week_flat, out_hbm,
                 item_v, t1_v, t2_v, midx_v, widx_v, sbuf,
                 sem_m, sem_w):
    wid = lax.axis_index("s") * NC + lax.axis_index("c")
    s0 = jnp.where(wid < 24, 6 * wid, 144 + 7 * (wid - 24))
    ns = jnp.where(wid < 24, 6, 7)

    pltpu.sync_copy(item_hbm.at[pl.ds(1024 * s0, 7168)], item_v)
    pltpu.sync_copy(t1_hbm.at[pl.ds(1024 * s0, 7168)], t1_v)
    pltpu.sync_copy(t2_hbm.at[pl.ds(1024 * s0, 7168)], t2_v)

    def slab(j, _):
        jb = 1024 * j

        def build(g, _):
            o = jb + 16 * g
            it16 = item_v[pl.ds(o, 16)]
            cbase = lax.shift_left(lax.shift_right_logical(it16, 7), 10) + (it16 & 127)
            mb16 = t1_v[pl.ds(o, 16)] * GSTRIDE + cbase
            wb16 = t2_v[pl.ds(o, 16)] * GSTRIDE + cbase
            vb = lax.shift_left(lax.shift_right_logical(g, 3), 10) + 16 * (g & 7)
            for dr in range(8):
                midx_v[pl.ds(vb + dr * 128, 16)] = mb16 + dr * 128
                widx_v[pl.ds(vb + dr * 128, 16)] = wb16 + dr * 128
            return 0

        lax.fori_loop(0, 64, build, 0)

        cpm = pltpu.make_async_copy(month_flat.at[midx_v], sbuf.at[pl.ds(0, 8192)], sem_m)
        cpw = pltpu.make_async_copy(week_flat.at[widx_v], sbuf.at[pl.ds(8192, 8192)], sem_w)
        cpm.start()
        cpw.start()
        cpm.wait()
        cpw.wait()

        pltpu.sync_copy(sbuf, out_hbm.at[pl.ds(16384 * (s0 + j), 16384)])
        return 0

    lax.fori_loop(0, ns, slab, 0)


@jax.jit
def _popularity_encode(item_flat, t1_flat, t2_flat, month_tbl, week_tbl,
                       mrem, wrem):
    mesh = plsc.VectorSubcoreMesh(core_axis_name="c", subcore_axis_name="s")
    retile = pl.kernel(
        _retile_body,
        out_type=(
            jax.ShapeDtypeStruct(((MROWS // 8) * GSTRIDE,), jnp.float32),
            jax.ShapeDtypeStruct(((WROWS // 8) * GSTRIDE,), jnp.float32),
        ),
        mesh=mesh,
        scratch_types=[
            pltpu.VMEM((8, BLK), jnp.float32),
            pltpu.VMEM((8, BLK), jnp.float32),
            pltpu.VMEM((BLK * 8,), jnp.float32),
            pltpu.VMEM((BLK * 8,), jnp.float32),
            pltpu.VMEM((8, 128), jnp.float32),
            pltpu.SemaphoreType.DMA,
            pltpu.SemaphoreType.DMA,
            pltpu.SemaphoreType.DMA,
            pltpu.SemaphoreType.DMA,
            pltpu.SemaphoreType.DMA,
        ],
        name="popularity_retile_sc",
    )
    month_flat, week_flat = retile(month_tbl, week_tbl, mrem, wrem)
    gather = pl.kernel(
        _gather_body,
        out_type=jax.ShapeDtypeStruct((N * D,), jnp.float32),
        mesh=mesh,
        scratch_types=[
            pltpu.VMEM((7168,), jnp.int32),
            pltpu.VMEM((7168,), jnp.int32),
            pltpu.VMEM((7168,), jnp.int32),
            pltpu.VMEM((8192,), jnp.int32),
            pltpu.VMEM((8192,), jnp.int32),
            pltpu.VMEM((16384,), jnp.float32),
            pltpu.SemaphoreType.DMA,
            pltpu.SemaphoreType.DMA,
        ],
        name="popularity_encoding_sc",
    )
    return gather(item_flat, t1_flat, t2_flat, month_flat, week_flat)


def kernel(log_seqs, time1_seqs, time2_seqs, month_pop_table, week_pop_table):
    item_flat = log_seqs.T.reshape(-1).astype(jnp.int32)
    t1_flat = time1_seqs.T.reshape(-1).astype(jnp.int32)
    t2_flat = time2_seqs.T.reshape(-1).astype(jnp.int32)
    pad = 128 - (W - 99968)
    mrem = jnp.pad(month_pop_table[:, 99968:], ((0, 0), (0, pad)))
    wrem = jnp.pad(week_pop_table[:, 99968:], ((0, 0), (0, pad)))
    out = _popularity_encode(item_flat, t1_flat, t2_flat,
                             month_pop_table, week_pop_table, mrem, wrem)
    return (out.reshape(L, 2, 8, 8, 128)
            .transpose(2, 4, 0, 1, 3)
            .reshape(B, L, D))
